# Initial kernel scaffold; baseline (speedup 1.0000x reference)
import jax
import jax.numpy as jnp
from jax.experimental import pallas as pl


def kernel(x, edge_index, Wl1, bl1, Wr1, br1, att1, bias1, g1, beta1, Wl2, bl2, Wr2, br2, att2, bias2, g2, beta2, Wl3, bl3, Wr3, br3, att3, bias3, g3, beta3):
    raise NotImplementedError("write your pallas kernel here")


# speedup vs baseline: 8.7971x; 8.7971x over previous
import functools

import jax
import jax.numpy as jnp
from jax import lax
from jax.experimental import pallas as pl
from jax.experimental.pallas import tpu as pltpu
from jax.experimental.pallas import tpu_sc as plsc

NN = 10000
D_IN_ = 128
HALF = 5120
NAH = HALF + 16
NT = 2 * HALF + 16
TPS = 16
NSC = 2
NW = NSC * TPS
RPT = NAH // TPS
EHAT = 320000 + NN
C = 128
NCH = 162
EPT = NCH * C
EPAD = TPS * EPT
RING = 160
DST_DISCARD = 1 << 20


@functools.cache
def _make_edge_kernel(dout):
    DC = dout // 16
    mesh = plsc.VectorSubcoreMesh(core_axis_name="c", subcore_axis_name="s")

    @functools.partial(
        pl.kernel,
        out_type=(
            jax.ShapeDtypeStruct((NSC, NAH, dout), jnp.float32),
            jax.ShapeDtypeStruct((NSC, TPS, NAH), jnp.float32),
        ),
        mesh=mesh,
        compiler_params=pltpu.CompilerParams(needs_layout_passes=False,
                                             use_tc_tiling_on_sc=False),
        scratch_types=(
            pltpu.VMEM((C,), jnp.int32),
            pltpu.VMEM((C,), jnp.int32),
            pltpu.VMEM((RING,), jnp.int32),
            pltpu.VMEM((RING,), jnp.int32),
            pltpu.VMEM((C,), jnp.int32),
            pltpu.VMEM((C,), jnp.int32),
            pltpu.VMEM((C,), jnp.int32),
            pltpu.VMEM((C, dout), jnp.float32),
            pltpu.VMEM((C, dout), jnp.float32),
            pltpu.VMEM((dout,), jnp.float32),
            pltpu.VMEM((NAH,), jnp.float32),
            pltpu.VMEM_SHARED((NAH, dout), jnp.float32),
            pltpu.SemaphoreType.DMA,
            pltpu.SemaphoreType.DMA,
        ),
    )
    def edge_kernel(src_hbm, dst_hbm, xl_hbm, xr_hbm, att_hbm,
                    out_hbm, den_hbm,
                    src_v, dst_v, ring_s, ring_d, psrc, pdst, pdst_g,
                    xl_rows, xr_rows, att_v, den_v, acc, sem1, sem2):
        c = lax.axis_index("c")
        s = lax.axis_index("s")
        chalf = c * HALF
        z16 = jnp.zeros((16,), jnp.float32)
        zi16 = jnp.zeros((16,), jnp.int32)
        lane = lax.iota(jnp.int32, 16)
        fill_d = jnp.full((16,), HALF, jnp.int32)

        def zrow(i, carry):
            for k in range(DC):
                xl_rows[i, pl.ds(16 * k, 16)] = z16
            return carry
        lax.fori_loop(0, C, zrow, 0)

        def zden(i, carry):
            den_v[pl.ds(i * 16, 16)] = z16
            return carry
        lax.fori_loop(0, NAH // 16, zden, 0)

        r0 = s * RPT
        pltpu.sync_copy(xl_rows, acc.at[pl.ds(r0, C)])
        pltpu.sync_copy(xl_rows, acc.at[pl.ds(r0 + C, C)])
        pltpu.sync_copy(xl_rows.at[pl.ds(0, RPT - 2 * C)],
                        acc.at[pl.ds(r0 + 2 * C, RPT - 2 * C)])
        pltpu.sync_copy(att_hbm, att_v)
        plsc.subcore_barrier()

        att_c = [att_v[pl.ds(16 * k, 16)] for k in range(DC)]
        gdn = lax.GatherDimensionNumbers(
            offset_dims=(), collapsed_slice_dims=(0,), start_index_map=(0,))
        rots = [((lane + sh) & 15) for sh in (8, 4, 2, 1)]

        def _sum16(v):
            for r in rots:
                v = v + lax.gather(v, r[:, None], gdn, slice_sizes=(1,),
                                   mode=lax.GatherScatterMode.PROMISE_IN_BOUNDS)
            return v

        def process():
            def bgidx(g, carry):
                psrc[pl.ds(16 * g, 16)] = ring_s[pl.ds(16 * g, 16)]
                dl = ring_d[pl.ds(16 * g, 16)]
                pdst[pl.ds(16 * g, 16)] = dl
                pdst_g[pl.ds(16 * g, 16)] = dl + chalf
                return carry
            lax.fori_loop(0, C // 16, bgidx, 0)

            cp1 = pltpu.async_copy(xl_hbm.at[psrc], xl_rows, sem1)
            cp2 = pltpu.async_copy(xr_hbm.at[pdst_g], xr_rows, sem2)
            cp1.wait()
            cp2.wait()

            def group_body(g, carry):
                dw = pdst[pl.ds(16 * g, 16)]

                def edge_body(l, carry2):
                    i = 16 * g + l
                    xlc = [xl_rows[i, pl.ds(16 * k, 16)] for k in range(DC)]
                    acc_e = z16
                    for k in range(DC):
                        v = xlc[k] + xr_rows[i, pl.ds(16 * k, 16)]
                        acc_e = acc_e + att_c[k] * jnp.maximum(v, 0.2 * v)
                    ex_v = jnp.exp(_sum16(acc_e))
                    plsc.addupdate_scatter(den_v, [dw], ex_v, mask=lane == l)
                    for k in range(DC):
                        xl_rows[i, pl.ds(16 * k, 16)] = xlc[k] * ex_v
                    return carry2
                lax.fori_loop(0, 16, edge_body, 0)
                return carry
            lax.fori_loop(0, C // 16, group_body, 0)

            pltpu.sync_copy(xl_rows, acc.at[pdst], add=True)

        def move_leftover():
            t1 = ring_s[pl.ds(C, 16)]
            ring_s[pl.ds(0, 16)] = t1
            t2 = ring_d[pl.ds(C, 16)]
            ring_d[pl.ds(0, 16)] = t2

        def chunk_body(j, ptr):
            base = s * EPT + j * C
            pltpu.sync_copy(src_hbm.at[pl.ds(base, C)], src_v)
            pltpu.sync_copy(dst_hbm.at[pl.ds(base, C)], dst_v)

            def group_c(g, p):
                sw = src_v[pl.ds(16 * g, 16)]
                dl = dst_v[pl.ds(16 * g, 16)] - chalf
                m = (dl >= 0) & (dl < HALF)
                plsc.store_compressed(ring_s.at[pl.ds(p, 16)], sw, mask=m)
                plsc.store_compressed(ring_d.at[pl.ds(p, 16)], dl, mask=m)
                p = p + plsc.all_reduce_population_count(m)[0]

                @pl.when(p >= C)
                def _():
                    process()
                    move_leftover()
                return jnp.where(p >= C, p - C, p)
            return lax.fori_loop(0, C // 16, group_c, ptr)
        ptr = lax.fori_loop(0, NCH, chunk_body, jnp.int32(0))

        for t in range(C // 16):
            @pl.when(ptr + 16 * t < C)
            def _():
                ring_s[pl.ds(ptr + 16 * t, 16)] = zi16
                ring_d[pl.ds(ptr + 16 * t, 16)] = fill_d
        process()

        plsc.subcore_barrier()
        pltpu.sync_copy(acc.at[pl.ds(r0, RPT)], out_hbm.at[c, pl.ds(r0, RPT)])
        pltpu.sync_copy(den_v, den_hbm.at[c, s])

    return edge_kernel


def _lin_body(x_ref, wl_ref, bl_ref, wr_ref, br_ref, xl_ref, xr_ref):
    x = x_ref[...]
    xl_ref[...] = (jnp.dot(x, wl_ref[...], preferred_element_type=jnp.float32)
                   + bl_ref[...][None, :])
    xr_ref[...] = (jnp.dot(x, wr_ref[...], preferred_element_type=jnp.float32)
                   + br_ref[...][None, :])


def _tc_lin(xp, Wl, bl, Wr, br):
    dout = Wl.shape[1]
    return pl.pallas_call(
        _lin_body,
        out_shape=(jax.ShapeDtypeStruct((NT, dout), jnp.float32),
                   jax.ShapeDtypeStruct((NT, dout), jnp.float32)),
    )(xp, Wl, bl, Wr, br)


def _norm_from_parts(acc_ref, den_ref, bias_ref, g_ref, beta_ref, relu, nrows):
    a = jnp.concatenate(
        [acc_ref[0, :HALF], acc_ref[1, :HALF], acc_ref[1, HALF:]], axis=0)[:nrows]
    d0 = jnp.sum(den_ref[0], axis=0)
    d1 = jnp.sum(den_ref[1], axis=0)
    den = jnp.concatenate([d0[:HALF], d1[:HALF], d1[HALF:]], axis=0)[:nrows]
    h = a / (den[:, None] + 1e-16) + bias_ref[...][None, :]
    mu = jnp.mean(h, axis=1, keepdims=True)
    var = jnp.mean((h - mu) ** 2, axis=1, keepdims=True)
    y = ((h - mu) / jnp.sqrt(var + 1e-5) * g_ref[...][None, :]
         + beta_ref[...][None, :])
    if relu:
        y = jnp.maximum(y, 0.0)
    return y


def _epi_lin_body(acc_ref, den_ref, bias_ref, g_ref, beta_ref,
                  wl_ref, bl_ref, wr_ref, br_ref, xl_ref, xr_ref):
    y = _norm_from_parts(acc_ref, den_ref, bias_ref, g_ref, beta_ref,
                         relu=True, nrows=NT)
    xl_ref[...] = (jnp.dot(y, wl_ref[...], preferred_element_type=jnp.float32)
                   + bl_ref[...][None, :])
    xr_ref[...] = (jnp.dot(y, wr_ref[...], preferred_element_type=jnp.float32)
                   + br_ref[...][None, :])


def _tc_epi_lin(acc, den, bias, g, beta, Wl, bl, Wr, br):
    dout = Wl.shape[1]
    return pl.pallas_call(
        _epi_lin_body,
        out_shape=(jax.ShapeDtypeStruct((NT, dout), jnp.float32),
                   jax.ShapeDtypeStruct((NT, dout), jnp.float32)),
    )(acc, den, bias, g, beta, Wl, bl, Wr, br)


def _epi_final_body(acc_ref, den_ref, bias_ref, g_ref, beta_ref, out_ref):
    out_ref[...] = _norm_from_parts(acc_ref, den_ref, bias_ref, g_ref,
                                    beta_ref, relu=False, nrows=NN)


def _tc_epi_final(acc, den, bias, g, beta):
    d = acc.shape[2]
    return pl.pallas_call(
        _epi_final_body,
        out_shape=jax.ShapeDtypeStruct((NN, d), jnp.float32),
    )(acc, den, bias, g, beta)


def kernel(x, edge_index,
           Wl1, bl1, Wr1, br1, att1, bias1, g1, beta1,
           Wl2, bl2, Wr2, br2, att2, bias2, g2, beta2,
           Wl3, bl3, Wr3, br3, att3, bias3, g3, beta3):
    xp = jnp.zeros((NT, D_IN_), jnp.float32).at[:NN].set(x)
    loop = jnp.arange(NN, dtype=jnp.int32)
    npad = EPAD - EHAT
    src = jnp.concatenate([edge_index[0].astype(jnp.int32), loop,
                           jnp.zeros((npad,), jnp.int32)])
    dst = jnp.concatenate([edge_index[1].astype(jnp.int32), loop,
                           jnp.full((npad,), DST_DISCARD, jnp.int32)])

    xl, xr = _tc_lin(xp, Wl1, bl1, Wr1, br1)
    acc, den = _make_edge_kernel(Wl1.shape[1])(src, dst, xl, xr, att1)
    xl, xr = _tc_epi_lin(acc, den, bias1, g1, beta1, Wl2, bl2, Wr2, br2)
    acc, den = _make_edge_kernel(Wl2.shape[1])(src, dst, xl, xr, att2)
    xl, xr = _tc_epi_lin(acc, den, bias2, g2, beta2, Wl3, bl3, Wr3, br3)
    acc, den = _make_edge_kernel(Wl3.shape[1])(src, dst, xl, xr, att3)
    return _tc_epi_final(acc, den, bias3, g3, beta3)


# speedup vs baseline: 10.3354x; 1.1749x over previous
import functools

import jax
import jax.numpy as jnp
from jax import lax
from jax.experimental import pallas as pl
from jax.experimental.pallas import tpu as pltpu
from jax.experimental.pallas import tpu_sc as plsc

NN = 10000
D_IN_ = 128
HALF = 5120
NAH = HALF + 16
NT = 2 * HALF + 16
TPS = 16
NSC = 2
NW = NSC * TPS
RPT = NAH // TPS
EHAT = 320000 + NN
C = 128
CR = 768
NCH = 27
EPT = NCH * CR
EPAD = TPS * EPT
RING = 160
DST_DISCARD = 1 << 20


@functools.cache
def _make_edge_kernel(dout):
    DC = dout // 16
    mesh = plsc.VectorSubcoreMesh(core_axis_name="c", subcore_axis_name="s")

    @functools.partial(
        pl.kernel,
        out_type=(
            jax.ShapeDtypeStruct((NSC, NAH, dout), jnp.float32),
            jax.ShapeDtypeStruct((NSC, TPS, NAH), jnp.float32),
        ),
        mesh=mesh,
        compiler_params=pltpu.CompilerParams(needs_layout_passes=False,
                                             use_tc_tiling_on_sc=False),
        scratch_types=(
            pltpu.VMEM((CR,), jnp.int32),
            pltpu.VMEM((CR,), jnp.int32),
            pltpu.VMEM((RING,), jnp.int32),
            pltpu.VMEM((RING,), jnp.int32),
            pltpu.VMEM((C,), jnp.int32),
            pltpu.VMEM((C,), jnp.int32),
            pltpu.VMEM((C,), jnp.int32),
            pltpu.VMEM((C, dout), jnp.float32),
            pltpu.VMEM((C, dout), jnp.float32),
            pltpu.VMEM((dout,), jnp.float32),
            pltpu.VMEM((NAH,), jnp.float32),
            pltpu.VMEM_SHARED((NAH, dout), jnp.float32),
            pltpu.SemaphoreType.DMA,
            pltpu.SemaphoreType.DMA,
        ),
    )
    def edge_kernel(src_hbm, dst_hbm, xl_hbm, xr_hbm, att_hbm,
                    out_hbm, den_hbm,
                    src_v, dst_v, ring_s, ring_d, psrc, pdst, pdst_g,
                    xl_rows, xr_rows, att_v, den_v, acc, sem1, sem2):
        c = lax.axis_index("c")
        s = lax.axis_index("s")
        chalf = c * HALF
        z16 = jnp.zeros((16,), jnp.float32)
        zi16 = jnp.zeros((16,), jnp.int32)
        lane = lax.iota(jnp.int32, 16)
        fill_d = jnp.full((16,), HALF, jnp.int32)

        def zrow(i, carry):
            for k in range(DC):
                xl_rows[i, pl.ds(16 * k, 16)] = z16
            return carry
        lax.fori_loop(0, C, zrow, 0)

        def zden(i, carry):
            den_v[pl.ds(i * 16, 16)] = z16
            return carry
        lax.fori_loop(0, NAH // 16, zden, 0)

        r0 = s * RPT
        pltpu.sync_copy(xl_rows, acc.at[pl.ds(r0, C)])
        pltpu.sync_copy(xl_rows, acc.at[pl.ds(r0 + C, C)])
        pltpu.sync_copy(xl_rows.at[pl.ds(0, RPT - 2 * C)],
                        acc.at[pl.ds(r0 + 2 * C, RPT - 2 * C)])
        pltpu.sync_copy(att_hbm, att_v)
        plsc.subcore_barrier()

        att_c = [att_v[pl.ds(16 * k, 16)] for k in range(DC)]
        gdn = lax.GatherDimensionNumbers(
            offset_dims=(), collapsed_slice_dims=(0,), start_index_map=(0,))
        rots = [((lane + sh) & 15) for sh in (8, 4, 2, 1)]

        def _sum16(v):
            for r in rots:
                v = v + lax.gather(v, r[:, None], gdn, slice_sizes=(1,),
                                   mode=lax.GatherScatterMode.PROMISE_IN_BOUNDS)
            return v

        def process():
            def bgidx(g, carry):
                psrc[pl.ds(16 * g, 16)] = ring_s[pl.ds(16 * g, 16)]
                dl = ring_d[pl.ds(16 * g, 16)]
                pdst[pl.ds(16 * g, 16)] = dl
                pdst_g[pl.ds(16 * g, 16)] = dl + chalf
                return carry
            lax.fori_loop(0, C // 16, bgidx, 0)

            cp1 = pltpu.async_copy(xl_hbm.at[psrc], xl_rows, sem1)
            cp2 = pltpu.async_copy(xr_hbm.at[pdst_g], xr_rows, sem2)
            cp1.wait()
            cp2.wait()

            masks = [lane == l for l in range(16)]

            def group_body(g, carry):
                dw = pdst[pl.ds(16 * g, 16)]
                for l in range(16):
                    i = 16 * g + l
                    xlc = [xl_rows[i, pl.ds(16 * k, 16)] for k in range(DC)]
                    acc_a = z16
                    acc_b = z16
                    for k in range(DC):
                        v = xlc[k] + xr_rows[i, pl.ds(16 * k, 16)]
                        t = att_c[k] * jnp.maximum(v, 0.2 * v)
                        if k % 2 == 0:
                            acc_a = acc_a + t
                        else:
                            acc_b = acc_b + t
                    ex_v = jnp.exp(_sum16(acc_a + acc_b))
                    plsc.addupdate_scatter(den_v, [dw], ex_v, mask=masks[l])
                    for k in range(DC):
                        xl_rows[i, pl.ds(16 * k, 16)] = xlc[k] * ex_v
                return carry
            lax.fori_loop(0, C // 16, group_body, 0)

            pltpu.sync_copy(xl_rows, acc.at[pdst], add=True)

        def move_leftover():
            t1 = ring_s[pl.ds(C, 16)]
            ring_s[pl.ds(0, 16)] = t1
            t2 = ring_d[pl.ds(C, 16)]
            ring_d[pl.ds(0, 16)] = t2

        def chunk_body(j, ptr):
            base = s * EPT + j * CR
            pltpu.sync_copy(src_hbm.at[pl.ds(base, CR)], src_v)
            pltpu.sync_copy(dst_hbm.at[pl.ds(base, CR)], dst_v)

            def group_c(g, p):
                sw = src_v[pl.ds(16 * g, 16)]
                dl = dst_v[pl.ds(16 * g, 16)] - chalf
                m = (dl >= 0) & (dl < HALF)
                plsc.store_compressed(ring_s.at[pl.ds(p, 16)], sw, mask=m)
                plsc.store_compressed(ring_d.at[pl.ds(p, 16)], dl, mask=m)
                p = p + plsc.all_reduce_population_count(m)[0]

                @pl.when(p >= C)
                def _():
                    process()
                    move_leftover()
                return jnp.where(p >= C, p - C, p)
            return lax.fori_loop(0, CR // 16, group_c, ptr)
        ptr = lax.fori_loop(0, NCH, chunk_body, jnp.int32(0))

        for t in range(C // 16):
            @pl.when(ptr + 16 * t < C)
            def _():
                ring_s[pl.ds(ptr + 16 * t, 16)] = zi16
                ring_d[pl.ds(ptr + 16 * t, 16)] = fill_d
        process()

        plsc.subcore_barrier()
        pltpu.sync_copy(acc.at[pl.ds(r0, RPT)], out_hbm.at[c, pl.ds(r0, RPT)])
        pltpu.sync_copy(den_v, den_hbm.at[c, s])

    return edge_kernel


def _lin_body(x_ref, wl_ref, bl_ref, wr_ref, br_ref, xl_ref, xr_ref):
    x = x_ref[...]
    xl_ref[...] = (jnp.dot(x, wl_ref[...], preferred_element_type=jnp.float32)
                   + bl_ref[...][None, :])
    xr_ref[...] = (jnp.dot(x, wr_ref[...], preferred_element_type=jnp.float32)
                   + br_ref[...][None, :])


def _tc_lin(xp, Wl, bl, Wr, br):
    dout = Wl.shape[1]
    return pl.pallas_call(
        _lin_body,
        out_shape=(jax.ShapeDtypeStruct((NT, dout), jnp.float32),
                   jax.ShapeDtypeStruct((NT, dout), jnp.float32)),
    )(xp, Wl, bl, Wr, br)


def _norm_from_parts(acc_ref, den_ref, bias_ref, g_ref, beta_ref, relu, nrows):
    a = jnp.concatenate(
        [acc_ref[0, :HALF], acc_ref[1, :HALF], acc_ref[1, HALF:]], axis=0)[:nrows]
    d0 = jnp.sum(den_ref[0], axis=0)
    d1 = jnp.sum(den_ref[1], axis=0)
    den = jnp.concatenate([d0[:HALF], d1[:HALF], d1[HALF:]], axis=0)[:nrows]
    h = a / (den[:, None] + 1e-16) + bias_ref[...][None, :]
    mu = jnp.mean(h, axis=1, keepdims=True)
    var = jnp.mean((h - mu) ** 2, axis=1, keepdims=True)
    y = ((h - mu) / jnp.sqrt(var + 1e-5) * g_ref[...][None, :]
         + beta_ref[...][None, :])
    if relu:
        y = jnp.maximum(y, 0.0)
    return y


def _epi_lin_body(acc_ref, den_ref, bias_ref, g_ref, beta_ref,
                  wl_ref, bl_ref, wr_ref, br_ref, xl_ref, xr_ref):
    y = _norm_from_parts(acc_ref, den_ref, bias_ref, g_ref, beta_ref,
                         relu=True, nrows=NT)
    xl_ref[...] = (jnp.dot(y, wl_ref[...], preferred_element_type=jnp.float32)
                   + bl_ref[...][None, :])
    xr_ref[...] = (jnp.dot(y, wr_ref[...], preferred_element_type=jnp.float32)
                   + br_ref[...][None, :])


def _tc_epi_lin(acc, den, bias, g, beta, Wl, bl, Wr, br):
    dout = Wl.shape[1]
    return pl.pallas_call(
        _epi_lin_body,
        out_shape=(jax.ShapeDtypeStruct((NT, dout), jnp.float32),
                   jax.ShapeDtypeStruct((NT, dout), jnp.float32)),
    )(acc, den, bias, g, beta, Wl, bl, Wr, br)


def _epi_final_body(acc_ref, den_ref, bias_ref, g_ref, beta_ref, out_ref):
    out_ref[...] = _norm_from_parts(acc_ref, den_ref, bias_ref, g_ref,
                                    beta_ref, relu=False, nrows=NN)


def _tc_epi_final(acc, den, bias, g, beta):
    d = acc.shape[2]
    return pl.pallas_call(
        _epi_final_body,
        out_shape=jax.ShapeDtypeStruct((NN, d), jnp.float32),
    )(acc, den, bias, g, beta)


def kernel(x, edge_index,
           Wl1, bl1, Wr1, br1, att1, bias1, g1, beta1,
           Wl2, bl2, Wr2, br2, att2, bias2, g2, beta2,
           Wl3, bl3, Wr3, br3, att3, bias3, g3, beta3):
    xp = jnp.zeros((NT, D_IN_), jnp.float32).at[:NN].set(x)
    loop = jnp.arange(NN, dtype=jnp.int32)
    npad = EPAD - EHAT
    src = jnp.concatenate([edge_index[0].astype(jnp.int32), loop,
                           jnp.zeros((npad,), jnp.int32)])
    dst = jnp.concatenate([edge_index[1].astype(jnp.int32), loop,
                           jnp.full((npad,), DST_DISCARD, jnp.int32)])

    xl, xr = _tc_lin(xp, Wl1, bl1, Wr1, br1)
    acc, den = _make_edge_kernel(Wl1.shape[1])(src, dst, xl, xr, att1)
    xl, xr = _tc_epi_lin(acc, den, bias1, g1, beta1, Wl2, bl2, Wr2, br2)
    acc, den = _make_edge_kernel(Wl2.shape[1])(src, dst, xl, xr, att2)
    xl, xr = _tc_epi_lin(acc, den, bias2, g2, beta2, Wl3, bl3, Wr3, br3)
    acc, den = _make_edge_kernel(Wl3.shape[1])(src, dst, xl, xr, att3)
    return _tc_epi_final(acc, den, bias3, g3, beta3)


# speedup vs baseline: 12.1630x; 1.1768x over previous
import functools

import jax
import jax.numpy as jnp
from jax import lax
from jax.experimental import pallas as pl
from jax.experimental.pallas import tpu as pltpu
from jax.experimental.pallas import tpu_sc as plsc

NN = 10000
D_IN_ = 128
HALF = 5120
NAH = HALF + 16
NT = 2 * HALF + 16
TPS = 16
NSC = 2
NW = NSC * TPS
RPT = NAH // TPS
EHAT = 320000 + NN
C = 128
CR = 768
NCH = 27
EPT = NCH * CR
EPAD = TPS * EPT
RING = 160
DST_DISCARD = 1 << 20


@functools.cache
def _make_edge_kernel(dout):
    DC = dout // 16
    C = 64 if dout > 160 else 128
    mesh = plsc.VectorSubcoreMesh(core_axis_name="c", subcore_axis_name="s")

    @functools.partial(
        pl.kernel,
        out_type=(
            jax.ShapeDtypeStruct((NSC, NAH, dout), jnp.float32),
            jax.ShapeDtypeStruct((NSC, TPS, NAH), jnp.float32),
        ),
        mesh=mesh,
        compiler_params=pltpu.CompilerParams(needs_layout_passes=False,
                                             use_tc_tiling_on_sc=False),
        scratch_types=(
            pltpu.VMEM((CR,), jnp.int32),
            pltpu.VMEM((CR,), jnp.int32),
            pltpu.VMEM((RING,), jnp.int32),
            pltpu.VMEM((RING,), jnp.int32),
            pltpu.VMEM((2, C), jnp.int32),
            pltpu.VMEM((2, C), jnp.int32),
            pltpu.VMEM((2, C + 16), jnp.int32),
            pltpu.VMEM((C,), jnp.int32),
            pltpu.VMEM((C,), jnp.int32),
            pltpu.VMEM((2, C, dout), jnp.float32),
            pltpu.VMEM((2, C, dout), jnp.float32),
            pltpu.VMEM((dout,), jnp.float32),
            pltpu.VMEM((NAH,), jnp.float32),
            pltpu.VMEM_SHARED((NAH, dout), jnp.float32),
            pltpu.SemaphoreType.DMA,
            pltpu.SemaphoreType.DMA,
            pltpu.SemaphoreType.DMA,
            pltpu.SemaphoreType.DMA,
        ),
    )
    def edge_kernel(src_hbm, dst_hbm, xl_hbm, xr_hbm, att_hbm,
                    out_hbm, den_hbm,
                    src_v, dst_v, ring_s, ring_d, psrc3, pdstg3, pdstw3,
                    pds0, pds1, xl3, xr3, att_v, den_v, acc,
                    sl0, sr0, sl1, sr1):
        c = lax.axis_index("c")
        s = lax.axis_index("s")
        chalf = c * HALF
        z16 = jnp.zeros((16,), jnp.float32)
        zi16 = jnp.zeros((16,), jnp.int32)
        lane = lax.iota(jnp.int32, 16)
        fill_d = jnp.full((16,), HALF, jnp.int32)

        def zrow(i, carry):
            for k in range(DC):
                xl3[0, i, pl.ds(16 * k, 16)] = z16
            return carry
        lax.fori_loop(0, C, zrow, 0)

        def zden(i, carry):
            den_v[pl.ds(i * 16, 16)] = z16
            return carry
        lax.fori_loop(0, NAH // 16, zden, 0)

        r0 = s * RPT
        for t in range(RPT // C):
            pltpu.sync_copy(xl3.at[0], acc.at[pl.ds(r0 + t * C, C)])
        rem = RPT % C
        if rem:
            pltpu.sync_copy(xl3.at[0, pl.ds(0, rem)],
                            acc.at[pl.ds(r0 + (RPT // C) * C, rem)])
        pltpu.sync_copy(att_hbm, att_v)
        plsc.subcore_barrier()

        att_c = [att_v[pl.ds(16 * k, 16)] for k in range(DC)]
        gdn = lax.GatherDimensionNumbers(
            offset_dims=(), collapsed_slice_dims=(0,), start_index_map=(0,))
        rots = [((lane + sh) & 15) for sh in (8, 4, 2, 1)]
        masks = [lane == l for l in range(8)]

        def _sum16(v):
            for r in rots:
                v = v + lax.gather(v, r[:, None], gdn, slice_sizes=(1,),
                                   mode=lax.GatherScatterMode.PROMISE_IN_BOUNDS)
            return v

        def stage_issue(b_):
            def cpg(g, carry):
                psrc3[b_, pl.ds(16 * g, 16)] = ring_s[pl.ds(16 * g, 16)]
                dl = ring_d[pl.ds(16 * g, 16)]
                pdstw3[b_, pl.ds(16 * g, 16)] = dl
                pdstg3[b_, pl.ds(16 * g, 16)] = dl + chalf
                return carry
            lax.fori_loop(0, C // 16, cpg, 0)

            def cps(tgt):
                def f(g, carry):
                    tgt[pl.ds(16 * g, 16)] = ring_d[pl.ds(16 * g, 16)]
                    return carry
                lax.fori_loop(0, C // 16, f, 0)

            def iss0():
                cps(pds0)
                pltpu.async_copy(xl_hbm.at[psrc3.at[0]], xl3.at[0], sl0)
                pltpu.async_copy(xr_hbm.at[pdstg3.at[0]], xr3.at[0], sr0)

            def iss1():
                cps(pds1)
                pltpu.async_copy(xl_hbm.at[psrc3.at[1]], xl3.at[1], sl1)
                pltpu.async_copy(xr_hbm.at[pdstg3.at[1]], xr3.at[1], sr1)
            lax.cond(b_ == 0, iss0, iss1)

        def compute_slot(b_):
            def group_body(g, carry):
                dw = pdstw3[b_, pl.ds(8 * g, 16)]
                for l in range(8):
                    i = 8 * g + l
                    xlc = [xl3[b_, i, pl.ds(16 * k, 16)] for k in range(DC)]
                    acc_a = z16
                    acc_b = z16
                    for k in range(DC):
                        v = xlc[k] + xr3[b_, i, pl.ds(16 * k, 16)]
                        t = att_c[k] * jnp.maximum(v, 0.2 * v)
                        if k % 2 == 0:
                            acc_a = acc_a + t
                        else:
                            acc_b = acc_b + t
                    ex_v = jnp.exp(_sum16(acc_a + acc_b))
                    plsc.addupdate_scatter(den_v, [dw], ex_v, mask=masks[l])
                    for k in range(DC):
                        xl3[b_, i, pl.ds(16 * k, 16)] = xlc[k] * ex_v
                return carry
            lax.fori_loop(0, C // 8, group_body, 0)

        def wait_slot(b_):
            def w0():
                pltpu.make_async_copy(xl_hbm.at[psrc3.at[0]], xl3.at[0], sl0).wait()
                pltpu.make_async_copy(xr_hbm.at[pdstg3.at[0]], xr3.at[0], sr0).wait()

            def w1():
                pltpu.make_async_copy(xl_hbm.at[psrc3.at[1]], xl3.at[1], sl1).wait()
                pltpu.make_async_copy(xr_hbm.at[pdstg3.at[1]], xr3.at[1], sr1).wait()
            lax.cond(b_ == 0, w0, w1)

        def finish_slot(b_):
            compute_slot(b_)

            def sc0():
                pltpu.sync_copy(xl3.at[0], acc.at[pds0], add=True)

            def sc1():
                pltpu.sync_copy(xl3.at[1], acc.at[pds1], add=True)
            lax.cond(b_ == 0, sc0, sc1)

        def move_leftover():
            t1 = ring_s[pl.ds(C, 16)]
            ring_s[pl.ds(0, 16)] = t1
            t2 = ring_d[pl.ds(C, 16)]
            ring_d[pl.ds(0, 16)] = t2

        def do_process(b_, pend_):
            @pl.when(pend_ == 1)
            def _():
                wait_slot(1 - b_)
            stage_issue(b_)

            @pl.when(pend_ == 1)
            def _():
                finish_slot(1 - b_)
            move_leftover()
            return 1 - b_, jnp.int32(1)

        def chunk_body(j, st):
            base = s * EPT + j * CR
            pltpu.sync_copy(src_hbm.at[pl.ds(base, CR)], src_v)
            pltpu.sync_copy(dst_hbm.at[pl.ds(base, CR)], dst_v)

            def group_c(g, st_):
                p, b_, pd = st_
                sw = src_v[pl.ds(16 * g, 16)]
                dl = dst_v[pl.ds(16 * g, 16)] - chalf
                m = (dl >= 0) & (dl < HALF)
                plsc.store_compressed(ring_s.at[pl.ds(p, 16)], sw, mask=m)
                plsc.store_compressed(ring_d.at[pl.ds(p, 16)], dl, mask=m)
                p = p + plsc.all_reduce_population_count(m)[0]
                b2, pd2 = lax.cond(p >= C, do_process,
                                   lambda bb, pp: (bb, pp), b_, pd)
                return jnp.where(p >= C, p - C, p), b2, pd2
            return lax.fori_loop(0, CR // 16, group_c, st)
        ptr, b, pend = lax.fori_loop(
            0, NCH, chunk_body, (jnp.int32(0), jnp.int32(0), jnp.int32(0)))

        for t in range(C // 16):
            @pl.when(ptr + 16 * t < C)
            def _():
                ring_s[pl.ds(ptr + 16 * t, 16)] = zi16
                ring_d[pl.ds(ptr + 16 * t, 16)] = fill_d
        b, pend = do_process(b, pend)
        wait_slot(1 - b)
        finish_slot(1 - b)

        plsc.subcore_barrier()
        pltpu.sync_copy(acc.at[pl.ds(r0, RPT)], out_hbm.at[c, pl.ds(r0, RPT)])
        pltpu.sync_copy(den_v, den_hbm.at[c, s])

    return edge_kernel


def _lin_body(x_ref, wl_ref, bl_ref, wr_ref, br_ref, xl_ref, xr_ref):
    x = x_ref[...]
    xl_ref[...] = (jnp.dot(x, wl_ref[...], preferred_element_type=jnp.float32)
                   + bl_ref[...][None, :])
    xr_ref[...] = (jnp.dot(x, wr_ref[...], preferred_element_type=jnp.float32)
                   + br_ref[...][None, :])


def _tc_lin(xp, Wl, bl, Wr, br):
    dout = Wl.shape[1]
    return pl.pallas_call(
        _lin_body,
        out_shape=(jax.ShapeDtypeStruct((NT, dout), jnp.float32),
                   jax.ShapeDtypeStruct((NT, dout), jnp.float32)),
    )(xp, Wl, bl, Wr, br)


def _norm_from_parts(acc_ref, den_ref, bias_ref, g_ref, beta_ref, relu, nrows):
    a = jnp.concatenate(
        [acc_ref[0, :HALF], acc_ref[1, :HALF], acc_ref[1, HALF:]], axis=0)[:nrows]
    d0 = jnp.sum(den_ref[0], axis=0)
    d1 = jnp.sum(den_ref[1], axis=0)
    den = jnp.concatenate([d0[:HALF], d1[:HALF], d1[HALF:]], axis=0)[:nrows]
    h = a / (den[:, None] + 1e-16) + bias_ref[...][None, :]
    mu = jnp.mean(h, axis=1, keepdims=True)
    var = jnp.mean((h - mu) ** 2, axis=1, keepdims=True)
    y = ((h - mu) / jnp.sqrt(var + 1e-5) * g_ref[...][None, :]
         + beta_ref[...][None, :])
    if relu:
        y = jnp.maximum(y, 0.0)
    return y


def _epi_lin_body(acc_ref, den_ref, bias_ref, g_ref, beta_ref,
                  wl_ref, bl_ref, wr_ref, br_ref, xl_ref, xr_ref):
    y = _norm_from_parts(acc_ref, den_ref, bias_ref, g_ref, beta_ref,
                         relu=True, nrows=NT)
    xl_ref[...] = (jnp.dot(y, wl_ref[...], preferred_element_type=jnp.float32)
                   + bl_ref[...][None, :])
    xr_ref[...] = (jnp.dot(y, wr_ref[...], preferred_element_type=jnp.float32)
                   + br_ref[...][None, :])


def _tc_epi_lin(acc, den, bias, g, beta, Wl, bl, Wr, br):
    dout = Wl.shape[1]
    return pl.pallas_call(
        _epi_lin_body,
        out_shape=(jax.ShapeDtypeStruct((NT, dout), jnp.float32),
                   jax.ShapeDtypeStruct((NT, dout), jnp.float32)),
    )(acc, den, bias, g, beta, Wl, bl, Wr, br)


def _epi_final_body(acc_ref, den_ref, bias_ref, g_ref, beta_ref, out_ref):
    out_ref[...] = _norm_from_parts(acc_ref, den_ref, bias_ref, g_ref,
                                    beta_ref, relu=False, nrows=NN)


def _tc_epi_final(acc, den, bias, g, beta):
    d = acc.shape[2]
    return pl.pallas_call(
        _epi_final_body,
        out_shape=jax.ShapeDtypeStruct((NN, d), jnp.float32),
    )(acc, den, bias, g, beta)


def kernel(x, edge_index,
           Wl1, bl1, Wr1, br1, att1, bias1, g1, beta1,
           Wl2, bl2, Wr2, br2, att2, bias2, g2, beta2,
           Wl3, bl3, Wr3, br3, att3, bias3, g3, beta3):
    xp = jnp.zeros((NT, D_IN_), jnp.float32).at[:NN].set(x)
    loop = jnp.arange(NN, dtype=jnp.int32)
    npad = EPAD - EHAT
    src = jnp.concatenate([edge_index[0].astype(jnp.int32), loop,
                           jnp.zeros((npad,), jnp.int32)])
    dst = jnp.concatenate([edge_index[1].astype(jnp.int32), loop,
                           jnp.full((npad,), DST_DISCARD, jnp.int32)])

    xl, xr = _tc_lin(xp, Wl1, bl1, Wr1, br1)
    acc, den = _make_edge_kernel(Wl1.shape[1])(src, dst, xl, xr, att1)
    xl, xr = _tc_epi_lin(acc, den, bias1, g1, beta1, Wl2, bl2, Wr2, br2)
    acc, den = _make_edge_kernel(Wl2.shape[1])(src, dst, xl, xr, att2)
    xl, xr = _tc_epi_lin(acc, den, bias2, g2, beta2, Wl3, bl3, Wr3, br3)
    acc, den = _make_edge_kernel(Wl3.shape[1])(src, dst, xl, xr, att3)
    return _tc_epi_final(acc, den, bias3, g3, beta3)
